# baseline (speedup 1.0000x reference)
import jax
import jax.numpy as jnp
from jax import lax
from jax.experimental import pallas as pl
from jax.experimental.pallas import tpu as pltpu
from jax.experimental.pallas import tpu_sc as plsc

_B, _C = 4096, 1000
_NC, _NS = 2, 16
_NW = _NC * _NS
_RW = _B // _NW
_CH = 16
_NCH = _RW // _CH


def _body(x_hbm, out_hbm, x_v, e_v, out_v, den2_v, e1_v, den1_v, e0_v):
    wid = lax.axis_index("s") * _NC + lax.axis_index("c")
    riota = lax.iota(jnp.int32, 16)
    zero = jnp.zeros((16,), jnp.float32)

    def g16(ref, col):
        return plsc.load_gather(ref, [riota, jnp.full((16,), col, jnp.int32)])

    def s16(ref, col, val):
        plsc.store_scatter(ref, [riota, jnp.full((16,), col, jnp.int32)], val)

    def chunk_body(ch, _carry):
        row0 = wid * _RW + ch * _CH
        pltpu.sync_copy(x_hbm.at[pl.ds(row0, _CH), :], x_v)

        def node_body(j, n0den):
            def group_body(k, carry):
                n1sum, n1den = carry
                g = j * 10 + k
                xsum = zero
                esum = zero
                for m in range(10):
                    c = g * 10 + m
                    v = g16(x_v, c)
                    e = jnp.exp(v)
                    s16(e_v, c, e)
                    xsum = xsum + v
                    esum = esum + e
                m2 = xsum * 0.1
                e1 = jnp.exp(m2)
                s16(den2_v, g, esum)
                s16(e1_v, g, e1)
                return (n1sum + m2, n1den + e1)

            n1sum, n1den = lax.fori_loop(0, 10, group_body, (zero, zero))
            m1 = n1sum * 0.1
            e0 = jnp.exp(m1)
            s16(den1_v, j, n1den)
            s16(e0_v, j, e0)
            return n0den + e0

        n0den = lax.fori_loop(0, 10, node_body, zero)

        def node2(j, _):
            f = g16(e0_v, j) / (n0den * g16(den1_v, j))

            def group2(k, __):
                g = j * 10 + k
                h = g16(e1_v, g) * f / g16(den2_v, g)
                for m in range(10):
                    c = g * 10 + m
                    s16(out_v, c, g16(e_v, c) * h)
                return 0

            return lax.fori_loop(0, 10, group2, 0)

        lax.fori_loop(0, 10, node2, 0)
        pltpu.sync_copy(out_v, out_hbm.at[pl.ds(row0, _CH), :])
        return 0

    lax.fori_loop(0, _NCH, chunk_body, 0)


def sc_kernel(outputs):
    call = pl.kernel(
        _body,
        out_type=jax.ShapeDtypeStruct((_B, _C), jnp.float32),
        mesh=plsc.VectorSubcoreMesh(
            core_axis_name="c", subcore_axis_name="s",
            num_cores=_NC, num_subcores=_NS,
        ),
        compiler_params=pltpu.CompilerParams(
            use_tc_tiling_on_sc=False, needs_layout_passes=False,
        ),
        scratch_types=[
            pltpu.VMEM((_CH, _C), jnp.float32),
            pltpu.VMEM((_CH, _C), jnp.float32),
            pltpu.VMEM((_CH, _C), jnp.float32),
            pltpu.VMEM((_CH, 100), jnp.float32),
            pltpu.VMEM((_CH, 100), jnp.float32),
            pltpu.VMEM((_CH, 16), jnp.float32),
            pltpu.VMEM((_CH, 16), jnp.float32),
        ],
    )
    return call(outputs)


def kernel(outputs):
    return sc_kernel(outputs)


# speedup vs baseline: 2.4891x; 2.4891x over previous
import jax
import jax.numpy as jnp
from jax import lax
from jax.experimental import pallas as pl
from jax.experimental.pallas import tpu as pltpu
from jax.experimental.pallas import tpu_sc as plsc

_B, _C = 4096, 1000
_G = 100
_N = 10
_BB = 512
_TC_ROWS = 3584
_SC_ROWS = _B - _TC_ROWS
_NC, _NS = 2, 16
_NW = _NC * _NS
_RW = _SC_ROWS // _NW
_CH = 16
_NCH = _RW // _CH


def _sel(rows, cols, div):
    r = jax.lax.broadcasted_iota(jnp.int32, (rows, cols), 0)
    c = jax.lax.broadcasted_iota(jnp.int32, (rows, cols), 1)
    return jnp.where(r // div == c, 1.0, 0.0).astype(jnp.float32)


def _bcast(rows, cols, div):
    r = jax.lax.broadcasted_iota(jnp.int32, (rows, cols), 0)
    c = jax.lax.broadcasted_iota(jnp.int32, (rows, cols), 1)
    return jnp.where(c // div == r, 1.0, 0.0).astype(jnp.float32)


def _tc_body(x_ref, o_ref):
    x = x_ref[...]
    e2 = jnp.exp(x)
    s10 = _sel(_C, _G, 10)
    s100 = _sel(_G, _N, 10)
    den2 = jnp.dot(e2, s10, preferred_element_type=jnp.float32)
    m2 = jnp.dot(x, s10, preferred_element_type=jnp.float32) * 0.1
    m2s = m2 - jnp.max(m2, axis=1, keepdims=True)
    e1 = jnp.exp(m2s)
    den1 = jnp.dot(e1, s100, preferred_element_type=jnp.float32)
    m1 = jnp.dot(m2s, s100, preferred_element_type=jnp.float32) * 0.1
    m1s = m1 - jnp.max(m1, axis=1, keepdims=True)
    em1 = jnp.exp(m1s)
    p0 = em1 / jnp.sum(em1, axis=1, keepdims=True)
    r10 = _bcast(_N, _G, 10)
    f = jnp.dot(p0 / den1, r10, preferred_element_type=jnp.float32)
    scale_g = e1 * f / den2
    r100 = _bcast(_G, _C, 10)
    h = jnp.dot(scale_g, r100, preferred_element_type=jnp.float32)
    o_ref[...] = e2 * h


def _tc_part(outputs):
    return pl.pallas_call(
        _tc_body,
        grid=(_TC_ROWS // _BB,),
        in_specs=[pl.BlockSpec((_BB, _C), lambda i: (i, 0))],
        out_specs=pl.BlockSpec((_BB, _C), lambda i: (i, 0)),
        out_shape=jax.ShapeDtypeStruct((_TC_ROWS, _C), outputs.dtype),
    )(outputs)


def _sc_body(x_hbm, out_hbm, x_v, e_v, out_v, den2_v, e1_v, den1_v, e0_v):
    wid = lax.axis_index("s") * _NC + lax.axis_index("c")
    riota = lax.iota(jnp.int32, 16)
    zero = jnp.zeros((16,), jnp.float32)

    def g16(ref, col):
        return plsc.load_gather(ref, [riota, jnp.full((16,), col, jnp.int32)])

    def s16(ref, col, val):
        plsc.store_scatter(ref, [riota, jnp.full((16,), col, jnp.int32)], val)

    def chunk_body(ch, _carry):
        row0 = _TC_ROWS + wid * _RW + ch * _CH
        pltpu.sync_copy(x_hbm.at[pl.ds(row0, _CH), :], x_v)

        def node_body(j, n0den):
            def group_body(k, carry):
                n1sum, n1den = carry
                g = j * 10 + k
                xsum = zero
                esum = zero
                for m in range(10):
                    c = g * 10 + m
                    v = g16(x_v, c)
                    e = jnp.exp(v)
                    s16(e_v, c, e)
                    xsum = xsum + v
                    esum = esum + e
                m2 = xsum * 0.1
                e1 = jnp.exp(m2)
                s16(den2_v, g, esum)
                s16(e1_v, g, e1)
                return (n1sum + m2, n1den + e1)

            n1sum, n1den = lax.fori_loop(0, 10, group_body, (zero, zero))
            m1 = n1sum * 0.1
            e0 = jnp.exp(m1)
            s16(den1_v, j, n1den)
            s16(e0_v, j, e0)
            return n0den + e0

        n0den = lax.fori_loop(0, 10, node_body, zero)

        def node2(j, _):
            f = g16(e0_v, j) / (n0den * g16(den1_v, j))

            def group2(k, __):
                g = j * 10 + k
                h = g16(e1_v, g) * f / g16(den2_v, g)
                for m in range(10):
                    c = g * 10 + m
                    s16(out_v, c, g16(e_v, c) * h)
                return 0

            return lax.fori_loop(0, 10, group2, 0)

        lax.fori_loop(0, 10, node2, 0)
        pltpu.sync_copy(out_v, out_hbm.at[pl.ds(row0 - _TC_ROWS, _CH), :])
        return 0

    lax.fori_loop(0, _NCH, chunk_body, 0)


def _sc_part(outputs):
    call = pl.kernel(
        _sc_body,
        out_type=jax.ShapeDtypeStruct((_SC_ROWS, _C), jnp.float32),
        mesh=plsc.VectorSubcoreMesh(
            core_axis_name="c", subcore_axis_name="s",
            num_cores=_NC, num_subcores=_NS,
        ),
        compiler_params=pltpu.CompilerParams(
            use_tc_tiling_on_sc=False, needs_layout_passes=False,
        ),
        scratch_types=[
            pltpu.VMEM((_CH, _C), jnp.float32),
            pltpu.VMEM((_CH, _C), jnp.float32),
            pltpu.VMEM((_CH, _C), jnp.float32),
            pltpu.VMEM((_CH, 100), jnp.float32),
            pltpu.VMEM((_CH, 100), jnp.float32),
            pltpu.VMEM((_CH, 16), jnp.float32),
            pltpu.VMEM((_CH, 16), jnp.float32),
        ],
    )
    return call(outputs)


def kernel(outputs):
    tc_out = _tc_part(outputs)
    sc_out = _sc_part(outputs)
    return jnp.concatenate([tc_out, sc_out], axis=0)


# speedup vs baseline: 5.5123x; 2.2146x over previous
import jax
import jax.numpy as jnp
from jax.experimental import pallas as pl

_C = 1000
_G = 100
_N = 10
_BB = 2048


def _sel(rows, cols, div):
    r = jax.lax.broadcasted_iota(jnp.int32, (rows, cols), 0)
    c = jax.lax.broadcasted_iota(jnp.int32, (rows, cols), 1)
    return jnp.where(r // div == c, 1.0, 0.0).astype(jnp.float32)


def _bcast(rows, cols, div):
    r = jax.lax.broadcasted_iota(jnp.int32, (rows, cols), 0)
    c = jax.lax.broadcasted_iota(jnp.int32, (rows, cols), 1)
    return jnp.where(c // div == r, 1.0, 0.0).astype(jnp.float32)


def _tree_softmax_kernel(x_ref, o_ref):
    x = x_ref[...]
    e2 = jnp.exp(x)
    s10 = _sel(_C, _G, 10)
    s100 = _sel(_G, _N, 10)
    den2 = jnp.dot(e2, s10, preferred_element_type=jnp.float32)
    m2 = jnp.dot(x, s10, preferred_element_type=jnp.float32) * 0.1
    m2s = m2 - jnp.max(m2, axis=1, keepdims=True)
    e1 = jnp.exp(m2s)
    den1 = jnp.dot(e1, s100, preferred_element_type=jnp.float32)
    m1 = jnp.dot(m2s, s100, preferred_element_type=jnp.float32) * 0.1
    m1s = m1 - jnp.max(m1, axis=1, keepdims=True)
    em1 = jnp.exp(m1s)
    p0 = em1 / jnp.sum(em1, axis=1, keepdims=True)
    r10 = _bcast(_N, _G, 10)
    f = jnp.dot(p0 / den1, r10, preferred_element_type=jnp.float32)
    scale_g = e1 * f / den2
    r100 = _bcast(_G, _C, 10)
    h = jnp.dot(scale_g, r100, preferred_element_type=jnp.float32)
    o_ref[...] = e2 * h


def kernel(outputs):
    b, c = outputs.shape
    return pl.pallas_call(
        _tree_softmax_kernel,
        grid=(b // _BB,),
        in_specs=[pl.BlockSpec((_BB, _C), lambda i: (i, 0))],
        out_specs=pl.BlockSpec((_BB, _C), lambda i: (i, 0)),
        out_shape=jax.ShapeDtypeStruct((b, c), outputs.dtype),
    )(outputs)


# speedup vs baseline: 5.5388x; 1.0048x over previous
import jax
import jax.numpy as jnp
from jax.experimental import pallas as pl

_C = 1000
_G = 100
_N = 10
_BB = 1024


def _sel(rows, cols, div):
    r = jax.lax.broadcasted_iota(jnp.int32, (rows, cols), 0)
    c = jax.lax.broadcasted_iota(jnp.int32, (rows, cols), 1)
    return jnp.where(r // div == c, 1.0, 0.0).astype(jnp.float32)


def _bcast(rows, cols, div):
    r = jax.lax.broadcasted_iota(jnp.int32, (rows, cols), 0)
    c = jax.lax.broadcasted_iota(jnp.int32, (rows, cols), 1)
    return jnp.where(c // div == r, 1.0, 0.0).astype(jnp.float32)


def _tree_softmax_kernel(x_ref, o_ref):
    x = x_ref[...] - jnp.max(x_ref[...], axis=1, keepdims=True)
    e2 = jnp.exp(x)
    s10 = _sel(_C, _G, 10)
    s100 = _sel(_G, _N, 10)
    den2 = jnp.dot(e2, s10, preferred_element_type=jnp.float32)
    m2 = jnp.dot(x, s10, preferred_element_type=jnp.float32) * 0.1
    m2s = m2 - jnp.max(m2, axis=1, keepdims=True)
    e1 = jnp.exp(m2s)
    den1 = jnp.dot(e1, s100, preferred_element_type=jnp.float32)
    m1 = jnp.dot(m2s, s100, preferred_element_type=jnp.float32) * 0.1
    m1s = m1 - jnp.max(m1, axis=1, keepdims=True)
    em1 = jnp.exp(m1s)
    p0 = em1 / jnp.sum(em1, axis=1, keepdims=True)
    r10 = _bcast(_N, _G, 10)
    f = jnp.dot(p0 / den1, r10, preferred_element_type=jnp.float32)
    scale_g = e1 * f / den2
    r100 = _bcast(_G, _C, 10)
    h = jnp.dot(scale_g, r100, preferred_element_type=jnp.float32)
    o_ref[...] = e2 * h


def kernel(outputs):
    b, c = outputs.shape
    return pl.pallas_call(
        _tree_softmax_kernel,
        grid=(b // _BB,),
        in_specs=[pl.BlockSpec((_BB, _C), lambda i: (i, 0))],
        out_specs=pl.BlockSpec((_BB, _C), lambda i: (i, 0)),
        out_shape=jax.ShapeDtypeStruct((b, c), outputs.dtype),
    )(outputs)
